# Initial kernel scaffold; baseline (speedup 1.0000x reference)
import jax
import jax.numpy as jnp
from jax.experimental import pallas as pl


def kernel(x, conv_W1, conv_b1, conv_W2, conv_b2, pool_W1, pool_b1, pool_W2, pool_b2, cluster_labels, edge_index, batch):
    raise NotImplementedError("write your pallas kernel here")


# speedup vs baseline: 5.4634x; 5.4634x over previous
import functools

import jax
import jax.numpy as jnp
from jax import lax
from jax.experimental import pallas as pl
from jax.experimental.pallas import tpu as pltpu
from jax.experimental.pallas import tpu_sc as plsc

N = 10000
E = 320000
D = 128
NUM_LAYERS = 2
NUM_CLUSTERS = 4
NUM_GRAPHS = 16

NT = 10240
CHUNK = 128
NWORK = 32
CPW = 80
EPAD = NWORK * CPW * CHUNK

_mesh = plsc.VectorSubcoreMesh(core_axis_name="c", subcore_axis_name="s")


@functools.partial(
    pl.kernel,
    out_type=jax.ShapeDtypeStruct((2 * NT, D), jnp.float32),
    mesh=_mesh,
    scratch_types=[
        pltpu.VMEM((CHUNK,), jnp.int32),
        pltpu.VMEM((CHUNK,), jnp.int32),
        pltpu.VMEM((CHUNK, D), jnp.float32),
        pltpu.VMEM_SHARED((NT, D), jnp.float32),
        pltpu.SemaphoreType.DMA,
    ],
)
def _sc_agg(h_hbm, src_hbm, dst_hbm, zero_hbm, out_hbm, si, di, rows, table, sem):
    cid = lax.axis_index("c")
    sid = lax.axis_index("s")
    w = sid * 2 + cid
    zr = NT // 16
    pltpu.sync_copy(zero_hbm.at[pl.ds(sid * zr, zr)], table.at[pl.ds(sid * zr, zr)])
    plsc.subcore_barrier()

    @pl.loop(0, CPW)
    def _(k):
        base = (w * CPW + k) * CHUNK
        pltpu.sync_copy(src_hbm.at[pl.ds(base, CHUNK)], si)
        pltpu.sync_copy(dst_hbm.at[pl.ds(base, CHUNK)], di)
        pltpu.async_copy(h_hbm.at[si], rows, sem).wait()
        pltpu.sync_copy(rows, table.at[di], add=True)

    plsc.subcore_barrier()
    orr = NT // 16
    pltpu.sync_copy(table.at[pl.ds(sid * orr, orr)],
                    out_hbm.at[pl.ds(cid * NT + sid * orr, orr)])


def _update_body(j, agg_ref, h_ref, lab_ref, w1_ref, b1_ref, w2_ref, b2_ref, out_ref):
    agg = agg_ref[0:N, :] + agg_ref[NT:NT + N, :]
    h = h_ref[...]
    z = agg + h
    hid = jnp.maximum(
        jnp.dot(z, w1_ref[...], preferred_element_type=jnp.float32) + b1_ref[...], 0.0)
    new = jnp.dot(hid, w2_ref[...], preferred_element_type=jnp.float32) + b2_ref[...]
    mask = lab_ref[...] == j
    out_ref[...] = jnp.where(mask, new, h)


def _tc_update(j, agg2, h, labels, W1, b1, W2, b2):
    return pl.pallas_call(
        functools.partial(_update_body, j),
        out_shape=jax.ShapeDtypeStruct((N, D), jnp.float32),
    )(agg2, h, labels, W1, b1, W2, b2)


def _pool_body(h_ref, batch_ref, w1_ref, b1_ref, w2_ref, b2_ref, out_ref):
    rows = lax.broadcasted_iota(jnp.int32, (NUM_GRAPHS, N), 0)
    onehot = (rows == batch_ref[...]).astype(jnp.float32)
    pooled = jnp.dot(onehot, h_ref[...], preferred_element_type=jnp.float32)
    hid = jnp.maximum(
        jnp.dot(pooled, w1_ref[...], preferred_element_type=jnp.float32) + b1_ref[...], 0.0)
    out_ref[...] = jnp.dot(hid, w2_ref[...], preferred_element_type=jnp.float32) + b2_ref[...]


def _pool(h, batch_row, W1, b1, W2, b2):
    return pl.pallas_call(
        _pool_body,
        out_shape=jax.ShapeDtypeStruct((NUM_GRAPHS, D), jnp.float32),
    )(h, batch_row, W1, b1, W2, b2)


def kernel(x, conv_W1, conv_b1, conv_W2, conv_b2,
           pool_W1, pool_b1, pool_W2, pool_b2,
           cluster_labels, edge_index, batch):
    src = edge_index[0].astype(jnp.int32)
    dst = edge_index[1].astype(jnp.int32)
    pad = EPAD - E
    ar = jnp.arange(pad, dtype=jnp.int32)
    srcp = jnp.concatenate([src, ar % N])
    dstp = jnp.concatenate([dst, N + ar % (NT - N)])
    zeros = jnp.zeros((NT, D), jnp.float32)
    labels = cluster_labels.astype(jnp.int32).reshape(N, 1)
    batch_row = batch.astype(jnp.int32).reshape(1, N)

    h = x
    for i in range(NUM_LAYERS):
        for j in range(NUM_CLUSTERS):
            idx = i * NUM_CLUSTERS + j
            agg2 = _sc_agg(h, srcp, dstp, zeros)
            h = _tc_update(j, agg2, h, labels,
                           conv_W1[idx], conv_b1[idx].reshape(1, D),
                           conv_W2[idx], conv_b2[idx].reshape(1, D))
    return _pool(h, batch_row, pool_W1, pool_b1.reshape(1, D),
                 pool_W2, pool_b2.reshape(1, D))


# speedup vs baseline: 11.5740x; 2.1184x over previous
import functools

import jax
import jax.numpy as jnp
from jax import lax
from jax.experimental import pallas as pl
from jax.experimental.pallas import tpu as pltpu
from jax.experimental.pallas import tpu_sc as plsc

N = 10000
E = 320000
D = 128
NUM_LAYERS = 2
NUM_CLUSTERS = 4
NUM_GRAPHS = 16

NT = 10240
CHUNK = 128
NWORK = 32
CPW = 80
EPAD = NWORK * CPW * CHUNK

_mesh = plsc.VectorSubcoreMesh(core_axis_name="c", subcore_axis_name="s")


@functools.partial(
    pl.kernel,
    out_type=jax.ShapeDtypeStruct((2 * NT, D), jnp.float32),
    mesh=_mesh,
    scratch_types=[
        pltpu.VMEM((CPW // 2, CHUNK), jnp.int32),
        pltpu.VMEM((CPW // 2, CHUNK), jnp.int32),
        pltpu.VMEM((CHUNK, D), jnp.float32),
        pltpu.VMEM((CHUNK, D), jnp.float32),
        pltpu.VMEM_SHARED((NT, D), jnp.float32),
        pltpu.SemaphoreType.DMA,
        pltpu.SemaphoreType.DMA,
    ],
)
def _sc_agg(h_hbm, src_hbm, dst_hbm, zero_hbm, out_hbm,
            si, di, rows0, rows1, table, sem0, sem1):
    cid = lax.axis_index("c")
    sid = lax.axis_index("s")
    w = sid * 2 + cid
    HALF = CPW // 2
    zr = NT // 16

    for stage in range(2):
        pltpu.sync_copy(src_hbm.at[w, pl.ds(stage * HALF, HALF)], si)
        pltpu.sync_copy(dst_hbm.at[w, pl.ds(stage * HALF, HALF)], di)
        pltpu.async_copy(h_hbm.at[si.at[0]], rows0, sem0)
        pltpu.async_copy(h_hbm.at[si.at[1]], rows1, sem1)
        if stage == 0:
            pltpu.sync_copy(zero_hbm.at[pl.ds(sid * zr, zr)],
                            table.at[pl.ds(sid * zr, zr)])
            plsc.subcore_barrier()

        @pl.loop(0, HALF // 2)
        def _(kk):
            k0 = kk * 2
            pltpu.make_async_copy(h_hbm.at[si.at[k0]], rows0, sem0).wait()
            pltpu.sync_copy(rows0, table.at[di.at[k0]], add=True)

            @pl.when(k0 + 2 < HALF)
            def _():
                pltpu.async_copy(h_hbm.at[si.at[k0 + 2]], rows0, sem0)

            pltpu.make_async_copy(h_hbm.at[si.at[k0 + 1]], rows1, sem1).wait()
            pltpu.sync_copy(rows1, table.at[di.at[k0 + 1]], add=True)

            @pl.when(k0 + 3 < HALF)
            def _():
                pltpu.async_copy(h_hbm.at[si.at[k0 + 3]], rows1, sem1)

    plsc.subcore_barrier()
    orr = NT // 16
    pltpu.sync_copy(table.at[pl.ds(sid * orr, orr)],
                    out_hbm.at[pl.ds(cid * NT + sid * orr, orr)])


def _update_body(j, agg_ref, h_ref, lab_ref, w1_ref, b1_ref, w2_ref, b2_ref, out_ref):
    agg = agg_ref[0:N, :] + agg_ref[NT:NT + N, :]
    h = h_ref[...]
    z = agg + h
    hid = jnp.maximum(
        jnp.dot(z, w1_ref[...], preferred_element_type=jnp.float32) + b1_ref[...], 0.0)
    new = jnp.dot(hid, w2_ref[...], preferred_element_type=jnp.float32) + b2_ref[...]
    mask = lab_ref[...] == j
    out_ref[...] = jnp.where(mask, new, h)


def _tc_update(j, agg2, h, labels, W1, b1, W2, b2):
    return pl.pallas_call(
        functools.partial(_update_body, j),
        out_shape=jax.ShapeDtypeStruct((N, D), jnp.float32),
    )(agg2, h, labels, W1, b1, W2, b2)


def _pool_body(h_ref, batch_ref, w1_ref, b1_ref, w2_ref, b2_ref, out_ref):
    rows = lax.broadcasted_iota(jnp.int32, (NUM_GRAPHS, N), 0)
    onehot = (rows == batch_ref[...]).astype(jnp.float32)
    pooled = jnp.dot(onehot, h_ref[...], preferred_element_type=jnp.float32)
    hid = jnp.maximum(
        jnp.dot(pooled, w1_ref[...], preferred_element_type=jnp.float32) + b1_ref[...], 0.0)
    out_ref[...] = jnp.dot(hid, w2_ref[...], preferred_element_type=jnp.float32) + b2_ref[...]


def _pool(h, batch_row, W1, b1, W2, b2):
    return pl.pallas_call(
        _pool_body,
        out_shape=jax.ShapeDtypeStruct((NUM_GRAPHS, D), jnp.float32),
    )(h, batch_row, W1, b1, W2, b2)


def kernel(x, conv_W1, conv_b1, conv_W2, conv_b2,
           pool_W1, pool_b1, pool_W2, pool_b2,
           cluster_labels, edge_index, batch):
    src = edge_index[0].astype(jnp.int32)
    dst = edge_index[1].astype(jnp.int32)
    pad = EPAD - E
    ar = jnp.arange(pad, dtype=jnp.int32)
    srcp = jnp.concatenate([src, ar % N]).reshape(NWORK, CPW, CHUNK)
    dstp = jnp.concatenate([dst, N + ar % (NT - N)]).reshape(NWORK, CPW, CHUNK)
    zeros = jnp.zeros((NT, D), jnp.float32)
    labels = cluster_labels.astype(jnp.int32).reshape(N, 1)
    batch_row = batch.astype(jnp.int32).reshape(1, N)

    h = x
    for i in range(NUM_LAYERS):
        for j in range(NUM_CLUSTERS):
            idx = i * NUM_CLUSTERS + j
            agg2 = _sc_agg(h, srcp, dstp, zeros)
            h = _tc_update(j, agg2, h, labels,
                           conv_W1[idx], conv_b1[idx].reshape(1, D),
                           conv_W2[idx], conv_b2[idx].reshape(1, D))
    return _pool(h, batch_row, pool_W1, pool_b1.reshape(1, D),
                 pool_W2, pool_b2.reshape(1, D))


# speedup vs baseline: 23.8085x; 2.0571x over previous
import dataclasses
import functools

import jax
import jax.numpy as jnp
from jax import lax
from jax.experimental import pallas as pl
from jax.experimental.pallas import tpu as pltpu
from jax.experimental.pallas import tpu_sc as plsc

N = 10000
E = 320000
D = 128
NUM_LAYERS = 2
NUM_CLUSTERS = 4
NUM_GRAPHS = 16

NT = 10240
CHUNK = 128
NWORK = 32
EW = E // NWORK
BLK = 2000
CAPC = 80
CAP = CAPC * CHUNK
NREG = NUM_CLUSTERS * NWORK

_mesh = plsc.VectorSubcoreMesh(core_axis_name="c", subcore_axis_name="s")

_cp = pltpu.CompilerParams()
if "needs_layout_passes" in pltpu.CompilerParams.__dataclass_fields__:
    _cp = dataclasses.replace(_cp, needs_layout_passes=False)


@functools.partial(
    pl.kernel,
    out_type=(
        jax.ShapeDtypeStruct((NREG * CAP,), jnp.int32),
        jax.ShapeDtypeStruct((NREG * CAP,), jnp.int32),
        jax.ShapeDtypeStruct((NWORK, 16), jnp.int32),
    ),
    mesh=_mesh,
    scratch_types=[
        pltpu.VMEM((BLK,), jnp.int32),
        pltpu.VMEM((BLK,), jnp.int32),
        pltpu.VMEM((N,), jnp.int32),
        pltpu.VMEM((CAP,), jnp.int32),
        pltpu.VMEM((CAP,), jnp.int32),
        pltpu.VMEM((CAP,), jnp.int32),
        pltpu.VMEM((CAP,), jnp.int32),
        pltpu.VMEM((CAP,), jnp.int32),
        pltpu.VMEM((CAP,), jnp.int32),
        pltpu.VMEM((CAP,), jnp.int32),
        pltpu.VMEM((CAP,), jnp.int32),
        pltpu.VMEM((16,), jnp.int32),
    ],
    compiler_params=_cp,
)
def _sc_part(src_hbm, dst_hbm, lab_hbm, bsrc_hbm, bdst_hbm, cnt_hbm,
             sv, dv, lab, b0s, b0d, b1s, b1d, b2s, b2d, b3s, b3d, cv):
    cid = lax.axis_index("c")
    sid = lax.axis_index("s")
    w = sid * 2 + cid
    bs = (b0s, b1s, b2s, b3s)
    bd = (b0d, b1d, b2d, b3d)
    pltpu.sync_copy(lab_hbm, lab)
    lane = lax.iota(jnp.int32, 16)

    cnt = (jnp.int32(0), jnp.int32(0), jnp.int32(0), jnp.int32(0))
    for b in range(EW // BLK):
        pltpu.sync_copy(src_hbm.at[pl.ds(w * EW + b * BLK, BLK)], sv)
        pltpu.sync_copy(dst_hbm.at[pl.ds(w * EW + b * BLK, BLK)], dv)

        def body(v, c4):
            s16 = sv[pl.ds(v * 16, 16)]
            d16 = dv[pl.ds(v * 16, 16)]
            k16 = plsc.load_gather(lab, [d16])
            out = []
            for j in range(NUM_CLUSTERS):
                m = k16 == j
                plsc.store_compressed(bs[j].at[pl.ds(c4[j], 16)], s16, mask=m)
                plsc.store_compressed(bd[j].at[pl.ds(c4[j], 16)], d16, mask=m)
                nj = jnp.max(plsc.all_reduce_population_count(m))
                out.append(c4[j] + nj)
            return tuple(out)

        cnt = lax.fori_loop(0, BLK // 16, body, cnt)

    cvec = jnp.zeros((16,), jnp.int32)
    for j in range(NUM_CLUSTERS):
        cj = cnt[j]
        for t in range(8):
            off = cj + t * 16
            pad = lane + (t * 16 + w * 128)
            bs[j][pl.ds(off, 16)] = pad % N
            bd[j][pl.ds(off, 16)] = N + pad % (NT - N)
        nch = (cj + CHUNK) // CHUNK
        cvec = jnp.where(lane == j, jnp.broadcast_to(nch, (16,)), cvec)
        base = (j * NWORK + w) * CAP
        nb = (cj + CHUNK + 1023) // 1024

        @pl.loop(0, nb)
        def _(bb):
            pltpu.sync_copy(bs[j].at[pl.ds(bb * 1024, 1024)],
                            bsrc_hbm.at[pl.ds(base + bb * 1024, 1024)])
            pltpu.sync_copy(bd[j].at[pl.ds(bb * 1024, 1024)],
                            bdst_hbm.at[pl.ds(base + bb * 1024, 1024)])

    cv[...] = cvec
    pltpu.sync_copy(cv, cnt_hbm.at[w])


def _make_sc_agg(j):
    @functools.partial(
        pl.kernel,
        out_type=jax.ShapeDtypeStruct((2 * NT, D), jnp.float32),
        mesh=_mesh,
        scratch_types=[
            pltpu.VMEM((CAPC // 2, CHUNK), jnp.int32),
            pltpu.VMEM((CAPC // 2, CHUNK), jnp.int32),
            pltpu.VMEM((CHUNK, D), jnp.float32),
            pltpu.VMEM((CHUNK, D), jnp.float32),
            pltpu.VMEM((16,), jnp.int32),
            pltpu.VMEM_SHARED((NT, D), jnp.float32),
            pltpu.SemaphoreType.DMA,
            pltpu.SemaphoreType.DMA,
        ],
        compiler_params=_cp,
    )
    def _sc_agg(h_hbm, bsrc_hbm, bdst_hbm, cnt_hbm, zero_hbm, out_hbm,
                si, di, rows0, rows1, cv, table, sem0, sem1):
        cid = lax.axis_index("c")
        sid = lax.axis_index("s")
        w = sid * 2 + cid
        HALF = CAPC // 2
        zr = NT // 16
        lane = lax.iota(jnp.int32, 16)

        pltpu.sync_copy(cnt_hbm.at[w], cv)
        nch = jnp.max(jnp.where(lane == j, cv[...], 0))
        rowbase = (j * NWORK + w) * CAPC

        pltpu.sync_copy(zero_hbm.at[pl.ds(sid * zr, zr)],
                        table.at[pl.ds(sid * zr, zr)])
        plsc.subcore_barrier()

        for s in range(2):
            cnt_s = jnp.clip(nch - s * HALF, 0, HALF)

            @pl.when(cnt_s > 0)
            def _():
                pltpu.sync_copy(bsrc_hbm.at[pl.ds(rowbase + s * HALF, HALF)], si)
                pltpu.sync_copy(bdst_hbm.at[pl.ds(rowbase + s * HALF, HALF)], di)
                pltpu.async_copy(h_hbm.at[si.at[0]], rows0, sem0)

                @pl.when(cnt_s > 1)
                def _():
                    pltpu.async_copy(h_hbm.at[si.at[1]], rows1, sem1)

                @pl.loop(0, cnt_s // 2)
                def _(kk):
                    k0 = kk * 2
                    pltpu.make_async_copy(h_hbm.at[si.at[k0]], rows0, sem0).wait()
                    pltpu.sync_copy(rows0, table.at[di.at[k0]], add=True)

                    @pl.when(k0 + 2 < cnt_s)
                    def _():
                        pltpu.async_copy(h_hbm.at[si.at[k0 + 2]], rows0, sem0)

                    pltpu.make_async_copy(h_hbm.at[si.at[k0 + 1]], rows1, sem1).wait()
                    pltpu.sync_copy(rows1, table.at[di.at[k0 + 1]], add=True)

                    @pl.when(k0 + 3 < cnt_s)
                    def _():
                        pltpu.async_copy(h_hbm.at[si.at[k0 + 3]], rows1, sem1)

                @pl.when(cnt_s % 2 == 1)
                def _():
                    klast = cnt_s - 1
                    pltpu.make_async_copy(h_hbm.at[si.at[klast]], rows0, sem0).wait()
                    pltpu.sync_copy(rows0, table.at[di.at[klast]], add=True)

        plsc.subcore_barrier()
        orr = NT // 16
        pltpu.sync_copy(table.at[pl.ds(sid * orr, orr)],
                        out_hbm.at[pl.ds(cid * NT + sid * orr, orr)])

    return _sc_agg


_SC_AGG = {j: _make_sc_agg(j) for j in range(NUM_CLUSTERS)}


def _update_body(j, agg_ref, h_ref, lab_ref, w1_ref, b1_ref, w2_ref, b2_ref, out_ref):
    agg = agg_ref[0:N, :] + agg_ref[NT:NT + N, :]
    h = h_ref[...]
    z = agg + h
    hid = jnp.maximum(
        jnp.dot(z, w1_ref[...], preferred_element_type=jnp.float32) + b1_ref[...], 0.0)
    new = jnp.dot(hid, w2_ref[...], preferred_element_type=jnp.float32) + b2_ref[...]
    mask = lab_ref[...] == j
    out_ref[...] = jnp.where(mask, new, h)


def _tc_update(j, agg2, h, labels, W1, b1, W2, b2):
    return pl.pallas_call(
        functools.partial(_update_body, j),
        out_shape=jax.ShapeDtypeStruct((N, D), jnp.float32),
    )(agg2, h, labels, W1, b1, W2, b2)


def _pool_body(h_ref, batch_ref, w1_ref, b1_ref, w2_ref, b2_ref, out_ref):
    rows = lax.broadcasted_iota(jnp.int32, (NUM_GRAPHS, N), 0)
    onehot = (rows == batch_ref[...]).astype(jnp.float32)
    pooled = jnp.dot(onehot, h_ref[...], preferred_element_type=jnp.float32)
    hid = jnp.maximum(
        jnp.dot(pooled, w1_ref[...], preferred_element_type=jnp.float32) + b1_ref[...], 0.0)
    out_ref[...] = jnp.dot(hid, w2_ref[...], preferred_element_type=jnp.float32) + b2_ref[...]


def _pool(h, batch_row, W1, b1, W2, b2):
    return pl.pallas_call(
        _pool_body,
        out_shape=jax.ShapeDtypeStruct((NUM_GRAPHS, D), jnp.float32),
    )(h, batch_row, W1, b1, W2, b2)


def kernel(x, conv_W1, conv_b1, conv_W2, conv_b2,
           pool_W1, pool_b1, pool_W2, pool_b2,
           cluster_labels, edge_index, batch):
    src = edge_index[0].astype(jnp.int32)
    dst = edge_index[1].astype(jnp.int32)
    lab1d = cluster_labels.astype(jnp.int32)
    zeros = jnp.zeros((NT, D), jnp.float32)
    labels = lab1d.reshape(N, 1)
    batch_row = batch.astype(jnp.int32).reshape(1, N)

    bsrc, bdst, cnts = _sc_part(src, dst, lab1d)
    bsrc2 = bsrc.reshape(NREG * CAPC, CHUNK)
    bdst2 = bdst.reshape(NREG * CAPC, CHUNK)

    h = x
    for i in range(NUM_LAYERS):
        for j in range(NUM_CLUSTERS):
            idx = i * NUM_CLUSTERS + j
            agg2 = _SC_AGG[j](h, bsrc2, bdst2, cnts, zeros)
            h = _tc_update(j, agg2, h, labels,
                           conv_W1[idx], conv_b1[idx].reshape(1, D),
                           conv_W2[idx], conv_b2[idx].reshape(1, D))
    return _pool(h, batch_row, pool_W1, pool_b1.reshape(1, D),
                 pool_W2, pool_b2.reshape(1, D))


# speedup vs baseline: 25.0274x; 1.0512x over previous
import dataclasses
import functools

import jax
import jax.numpy as jnp
from jax import lax
from jax.experimental import pallas as pl
from jax.experimental.pallas import tpu as pltpu
from jax.experimental.pallas import tpu_sc as plsc

N = 10000
E = 320000
D = 128
NUM_LAYERS = 2
NUM_CLUSTERS = 4
NUM_GRAPHS = 16

NT = 10240
CHUNK = 128
NWORK = 32
EW = E // NWORK
BLK = 2000
CAPC = 80
CAP = CAPC * CHUNK
NREG = NUM_CLUSTERS * NWORK

_mesh = plsc.VectorSubcoreMesh(core_axis_name="c", subcore_axis_name="s")

_cp = pltpu.CompilerParams()
if "needs_layout_passes" in pltpu.CompilerParams.__dataclass_fields__:
    _cp = dataclasses.replace(_cp, needs_layout_passes=False)


@functools.partial(
    pl.kernel,
    out_type=(
        jax.ShapeDtypeStruct((NREG * CAP,), jnp.int32),
        jax.ShapeDtypeStruct((NREG * CAP,), jnp.int32),
        jax.ShapeDtypeStruct((NWORK, 16), jnp.int32),
    ),
    mesh=_mesh,
    scratch_types=[
        pltpu.VMEM((BLK,), jnp.int32),
        pltpu.VMEM((BLK,), jnp.int32),
        pltpu.VMEM((N,), jnp.int32),
        pltpu.VMEM((CAP,), jnp.int32),
        pltpu.VMEM((CAP,), jnp.int32),
        pltpu.VMEM((CAP,), jnp.int32),
        pltpu.VMEM((CAP,), jnp.int32),
        pltpu.VMEM((CAP,), jnp.int32),
        pltpu.VMEM((CAP,), jnp.int32),
        pltpu.VMEM((CAP,), jnp.int32),
        pltpu.VMEM((CAP,), jnp.int32),
        pltpu.VMEM((16,), jnp.int32),
    ],
    compiler_params=_cp,
)
def _sc_part(src_hbm, dst_hbm, lab_hbm, bsrc_hbm, bdst_hbm, cnt_hbm,
             sv, dv, lab, b0s, b0d, b1s, b1d, b2s, b2d, b3s, b3d, cv):
    cid = lax.axis_index("c")
    sid = lax.axis_index("s")
    w = sid * 2 + cid
    bs = (b0s, b1s, b2s, b3s)
    bd = (b0d, b1d, b2d, b3d)
    pltpu.sync_copy(lab_hbm, lab)
    lane = lax.iota(jnp.int32, 16)

    cnt = (jnp.int32(0), jnp.int32(0), jnp.int32(0), jnp.int32(0))
    for b in range(EW // BLK):
        pltpu.sync_copy(src_hbm.at[pl.ds(w * EW + b * BLK, BLK)], sv)
        pltpu.sync_copy(dst_hbm.at[pl.ds(w * EW + b * BLK, BLK)], dv)

        def body(v, c4):
            s16 = sv[pl.ds(v * 16, 16)]
            d16 = dv[pl.ds(v * 16, 16)]
            k16 = plsc.load_gather(lab, [d16])
            out = []
            for j in range(NUM_CLUSTERS):
                m = k16 == j
                plsc.store_compressed(bs[j].at[pl.ds(c4[j], 16)], s16, mask=m)
                plsc.store_compressed(bd[j].at[pl.ds(c4[j], 16)], d16, mask=m)
                nj = jnp.max(plsc.all_reduce_population_count(m))
                out.append(c4[j] + nj)
            return tuple(out)

        cnt = lax.fori_loop(0, BLK // 16, body, cnt)

    cvec = jnp.zeros((16,), jnp.int32)
    for j in range(NUM_CLUSTERS):
        cj = cnt[j]
        for t in range(8):
            off = cj + t * 16
            pad = lane + (t * 16 + w * 128)
            bs[j][pl.ds(off, 16)] = pad % N
            bd[j][pl.ds(off, 16)] = N + pad % (NT - N)
        nch = (cj + CHUNK) // CHUNK
        cvec = jnp.where(lane == j, jnp.broadcast_to(nch, (16,)), cvec)
        base = (j * NWORK + w) * CAP
        nb = (cj + CHUNK + 1023) // 1024

        @pl.loop(0, nb)
        def _(bb):
            pltpu.sync_copy(bs[j].at[pl.ds(bb * 1024, 1024)],
                            bsrc_hbm.at[pl.ds(base + bb * 1024, 1024)])
            pltpu.sync_copy(bd[j].at[pl.ds(bb * 1024, 1024)],
                            bdst_hbm.at[pl.ds(base + bb * 1024, 1024)])

    cv[...] = cvec
    pltpu.sync_copy(cv, cnt_hbm.at[w])


def _make_sc_agg(j):
    @functools.partial(
        pl.kernel,
        out_type=jax.ShapeDtypeStruct((2 * NT, D), jnp.float32),
        mesh=_mesh,
        scratch_types=[
            pltpu.VMEM((CAPC // 2, CHUNK), jnp.int32),
            pltpu.VMEM((CAPC // 2, CHUNK), jnp.int32),
            pltpu.VMEM((CHUNK, D), jnp.float32),
            pltpu.VMEM((CHUNK, D), jnp.float32),
            pltpu.VMEM((16,), jnp.int32),
            pltpu.VMEM_SHARED((NT, D), jnp.float32),
            pltpu.SemaphoreType.DMA,
            pltpu.SemaphoreType.DMA,
        ],
        compiler_params=_cp,
    )
    def _sc_agg(h_hbm, bsrc_hbm, bdst_hbm, cnt_hbm, out_hbm,
                si, di, rows0, rows1, cv, table, sem0, sem1):
        cid = lax.axis_index("c")
        sid = lax.axis_index("s")
        w = sid * 2 + cid
        HALF = CAPC // 2
        zr = NT // 16
        lane = lax.iota(jnp.int32, 16)

        pltpu.sync_copy(cnt_hbm.at[w], cv)
        nch = jnp.max(jnp.where(lane == j, cv[...], 0))
        rowbase = (j * NWORK + w) * CAPC

        z16 = jnp.zeros((16,), jnp.float32)

        @pl.loop(0, CHUNK)
        def _(r):
            for t in range(D // 16):
                rows0[r, pl.ds(t * 16, 16)] = z16

        for t in range(zr // CHUNK):
            pltpu.sync_copy(rows0, table.at[pl.ds(sid * zr + t * CHUNK, CHUNK)])
        plsc.subcore_barrier()

        for s in range(2):
            cnt_s = jnp.clip(nch - s * HALF, 0, HALF)

            @pl.when(cnt_s > 0)
            def _():
                pltpu.sync_copy(bsrc_hbm.at[pl.ds(rowbase + s * HALF, HALF)], si)
                pltpu.sync_copy(bdst_hbm.at[pl.ds(rowbase + s * HALF, HALF)], di)
                pltpu.async_copy(h_hbm.at[si.at[0]], rows0, sem0)

                @pl.when(cnt_s > 1)
                def _():
                    pltpu.async_copy(h_hbm.at[si.at[1]], rows1, sem1)

                @pl.loop(0, cnt_s // 2)
                def _(kk):
                    k0 = kk * 2
                    pltpu.make_async_copy(h_hbm.at[si.at[k0]], rows0, sem0).wait()
                    pltpu.sync_copy(rows0, table.at[di.at[k0]], add=True)

                    @pl.when(k0 + 2 < cnt_s)
                    def _():
                        pltpu.async_copy(h_hbm.at[si.at[k0 + 2]], rows0, sem0)

                    pltpu.make_async_copy(h_hbm.at[si.at[k0 + 1]], rows1, sem1).wait()
                    pltpu.sync_copy(rows1, table.at[di.at[k0 + 1]], add=True)

                    @pl.when(k0 + 3 < cnt_s)
                    def _():
                        pltpu.async_copy(h_hbm.at[si.at[k0 + 3]], rows1, sem1)

                @pl.when(cnt_s % 2 == 1)
                def _():
                    klast = cnt_s - 1
                    pltpu.make_async_copy(h_hbm.at[si.at[klast]], rows0, sem0).wait()
                    pltpu.sync_copy(rows0, table.at[di.at[klast]], add=True)

        plsc.subcore_barrier()
        orr = NT // 16
        pltpu.sync_copy(table.at[pl.ds(sid * orr, orr)],
                        out_hbm.at[pl.ds(cid * NT + sid * orr, orr)])

    return _sc_agg


_SC_AGG = {j: _make_sc_agg(j) for j in range(NUM_CLUSTERS)}


def _update_body(j, agg_ref, h_ref, lab_ref, w1_ref, b1_ref, w2_ref, b2_ref, out_ref):
    agg = agg_ref[0:N, :] + agg_ref[NT:NT + N, :]
    h = h_ref[...]
    z = agg + h
    hid = jnp.maximum(
        jnp.dot(z, w1_ref[...], preferred_element_type=jnp.float32) + b1_ref[...], 0.0)
    new = jnp.dot(hid, w2_ref[...], preferred_element_type=jnp.float32) + b2_ref[...]
    mask = lab_ref[...] == j
    out_ref[...] = jnp.where(mask, new, h)


def _tc_update(j, agg2, h, labels, W1, b1, W2, b2):
    return pl.pallas_call(
        functools.partial(_update_body, j),
        out_shape=jax.ShapeDtypeStruct((N, D), jnp.float32),
    )(agg2, h, labels, W1, b1, W2, b2)


def _pool_body(h_ref, batch_ref, w1_ref, b1_ref, w2_ref, b2_ref, out_ref):
    rows = lax.broadcasted_iota(jnp.int32, (NUM_GRAPHS, N), 0)
    onehot = (rows == batch_ref[...]).astype(jnp.float32)
    pooled = jnp.dot(onehot, h_ref[...], preferred_element_type=jnp.float32)
    hid = jnp.maximum(
        jnp.dot(pooled, w1_ref[...], preferred_element_type=jnp.float32) + b1_ref[...], 0.0)
    out_ref[...] = jnp.dot(hid, w2_ref[...], preferred_element_type=jnp.float32) + b2_ref[...]


def _pool(h, batch_row, W1, b1, W2, b2):
    return pl.pallas_call(
        _pool_body,
        out_shape=jax.ShapeDtypeStruct((NUM_GRAPHS, D), jnp.float32),
    )(h, batch_row, W1, b1, W2, b2)


def kernel(x, conv_W1, conv_b1, conv_W2, conv_b2,
           pool_W1, pool_b1, pool_W2, pool_b2,
           cluster_labels, edge_index, batch):
    src = edge_index[0].astype(jnp.int32)
    dst = edge_index[1].astype(jnp.int32)
    lab1d = cluster_labels.astype(jnp.int32)
    labels = lab1d.reshape(N, 1)
    batch_row = batch.astype(jnp.int32).reshape(1, N)

    bsrc, bdst, cnts = _sc_part(src, dst, lab1d)
    bsrc2 = bsrc.reshape(NREG * CAPC, CHUNK)
    bdst2 = bdst.reshape(NREG * CAPC, CHUNK)

    h = x
    for i in range(NUM_LAYERS):
        for j in range(NUM_CLUSTERS):
            idx = i * NUM_CLUSTERS + j
            agg2 = _SC_AGG[j](h, bsrc2, bdst2, cnts)
            h = _tc_update(j, agg2, h, labels,
                           conv_W1[idx], conv_b1[idx].reshape(1, D),
                           conv_W2[idx], conv_b2[idx].reshape(1, D))
    return _pool(h, batch_row, pool_W1, pool_b1.reshape(1, D),
                 pool_W2, pool_b2.reshape(1, D))


# speedup vs baseline: 25.5743x; 1.0219x over previous
import dataclasses
import functools

import jax
import jax.numpy as jnp
from jax import lax
from jax.experimental import pallas as pl
from jax.experimental.pallas import tpu as pltpu
from jax.experimental.pallas import tpu_sc as plsc

N = 10000
E = 320000
D = 128
NUM_LAYERS = 2
NUM_CLUSTERS = 4
NUM_GRAPHS = 16

H0 = 5120
TH = 5248
CHUNK = 128
NWORK = 32
EW = E // NWORK
BLK = 2000
CAPC = 80
CAP = CAPC * CHUNK
NBKT = 2 * NUM_CLUSTERS
NREG = NBKT * NWORK
PSH = 8192

_mesh = plsc.VectorSubcoreMesh(core_axis_name="c", subcore_axis_name="s")

_cp = pltpu.CompilerParams()
if "needs_layout_passes" in pltpu.CompilerParams.__dataclass_fields__:
    _cp = dataclasses.replace(_cp, needs_layout_passes=False)


@functools.partial(
    pl.kernel,
    out_type=(
        jax.ShapeDtypeStruct((NREG * CAP,), jnp.int32),
        jax.ShapeDtypeStruct((NWORK, 16), jnp.int32),
    ),
    mesh=_mesh,
    scratch_types=[
        pltpu.VMEM((BLK,), jnp.int32),
        pltpu.VMEM((BLK,), jnp.int32),
        pltpu.VMEM((N,), jnp.int32),
        pltpu.VMEM((CAP,), jnp.int32),
        pltpu.VMEM((CAP,), jnp.int32),
        pltpu.VMEM((CAP,), jnp.int32),
        pltpu.VMEM((CAP,), jnp.int32),
        pltpu.VMEM((CAP,), jnp.int32),
        pltpu.VMEM((CAP,), jnp.int32),
        pltpu.VMEM((CAP,), jnp.int32),
        pltpu.VMEM((CAP,), jnp.int32),
        pltpu.VMEM((16,), jnp.int32),
    ],
    compiler_params=_cp,
)
def _sc_part(src_hbm, dst_hbm, lab_hbm, bp_hbm, cnt_hbm,
             sv, dv, lab, b0, b1, b2, b3, b4, b5, b6, b7, cv):
    cid = lax.axis_index("c")
    sid = lax.axis_index("s")
    w = sid * 2 + cid
    bb = (b0, b1, b2, b3, b4, b5, b6, b7)
    pltpu.sync_copy(lab_hbm, lab)
    lane = lax.iota(jnp.int32, 16)

    cnt = tuple(jnp.int32(0) for _ in range(NBKT))
    for blk in range(EW // BLK):
        pltpu.sync_copy(src_hbm.at[pl.ds(w * EW + blk * BLK, BLK)], sv)
        pltpu.sync_copy(dst_hbm.at[pl.ds(w * EW + blk * BLK, BLK)], dv)

        def body(v, c8):
            s16 = sv[pl.ds(v * 16, 16)]
            d16 = dv[pl.ds(v * 16, 16)]
            k16 = plsc.load_gather(lab, [d16])
            h16 = (d16 >= H0).astype(jnp.int32)
            b16 = k16 * 2 + h16
            p16 = s16 * PSH + (d16 - h16 * H0)
            out = []
            for b in range(NBKT):
                m = b16 == b
                plsc.store_compressed(bb[b].at[pl.ds(c8[b], 16)], p16, mask=m)
                nb_ = jnp.max(plsc.all_reduce_population_count(m))
                out.append(c8[b] + nb_)
            return tuple(out)

        cnt = lax.fori_loop(0, BLK // 16, body, cnt)

    cvec = jnp.zeros((16,), jnp.int32)
    for b in range(NBKT):
        cb = cnt[b]
        for t in range(8):
            off = cb + t * 16
            pad = lane + (t * 16 + w * 128)
            bb[b][pl.ds(off, 16)] = (pad % N) * PSH + H0 + pad % (TH - H0)
        nch = (cb + CHUNK) // CHUNK
        cvec = jnp.where(lane == b, jnp.broadcast_to(nch, (16,)), cvec)
        base = (b * NWORK + w) * CAP
        nblk = (cb + CHUNK + 1023) // 1024

        @pl.loop(0, nblk)
        def _(q):
            pltpu.sync_copy(bb[b].at[pl.ds(q * 1024, 1024)],
                            bp_hbm.at[pl.ds(base + q * 1024, 1024)])

    cv[...] = cvec
    pltpu.sync_copy(cv, cnt_hbm.at[w])


def _make_sc_agg(j):
    @functools.partial(
        pl.kernel,
        out_type=jax.ShapeDtypeStruct((2 * TH, D), jnp.float32),
        mesh=_mesh,
        scratch_types=[
            pltpu.VMEM((CAP // 2,), jnp.int32),
            pltpu.VMEM((2, CHUNK), jnp.int32),
            pltpu.VMEM((2, CHUNK), jnp.int32),
            pltpu.VMEM((CHUNK, D), jnp.float32),
            pltpu.VMEM((CHUNK, D), jnp.float32),
            pltpu.VMEM((16,), jnp.int32),
            pltpu.VMEM((16,), jnp.int32),
            pltpu.VMEM_SHARED((TH, D), jnp.float32),
            pltpu.SemaphoreType.DMA,
            pltpu.SemaphoreType.DMA,
        ],
        compiler_params=_cp,
    )
    def _sc_agg(h_hbm, bp_hbm, cnt_hbm, out_hbm,
                pb, si2, di2, rows0, rows1, cv0, cv1, table, sem0, sem1):
        cid = lax.axis_index("c")
        sid = lax.axis_index("s")
        HALF = CAPC // 2
        b = j * 2 + cid
        lane = lax.iota(jnp.int32, 16)
        zr = TH // 16

        z16 = jnp.zeros((16,), jnp.float32)

        @pl.loop(0, CHUNK)
        def _(r):
            for t in range(D // 16):
                rows0[r, pl.ds(t * 16, 16)] = z16

        pltpu.sync_copy(rows0, table.at[pl.ds(sid * zr, CHUNK)])
        pltpu.sync_copy(rows0, table.at[pl.ds(sid * zr + CHUNK, CHUNK)])
        pltpu.sync_copy(rows0.at[pl.ds(0, zr - 2 * CHUNK)],
                        table.at[pl.ds(sid * zr + 2 * CHUNK, zr - 2 * CHUNK)])
        plsc.subcore_barrier()

        def unpack(k, slot):
            for t in range(CHUNK // 16):
                pv = pb[pl.ds(k * CHUNK + t * 16, 16)]
                si2.at[slot][pl.ds(t * 16, 16)] = lax.shift_right_logical(
                    pv, jnp.int32(13))
                di2.at[slot][pl.ds(t * 16, 16)] = lax.bitwise_and(
                    pv, jnp.int32(PSH - 1))

        for r2 in range(2):
            r = sid * 2 + r2
            cvr = (cv0, cv1)[r2]
            pltpu.sync_copy(cnt_hbm.at[r], cvr)
            nch = jnp.max(jnp.where(lane == b, cvr[...], 0))
            base = (b * NWORK + r) * CAP

            for s in range(2):
                cnt_s = jnp.clip(nch - s * HALF, 0, HALF)

                @pl.when(cnt_s > 0)
                def _():
                    pltpu.sync_copy(bp_hbm.at[pl.ds(base + s * (CAP // 2),
                                                    CAP // 2)], pb)
                    unpack(0, 0)
                    pltpu.async_copy(h_hbm.at[si2.at[0]], rows0, sem0)

                    @pl.when(cnt_s > 1)
                    def _():
                        unpack(1, 1)
                        pltpu.async_copy(h_hbm.at[si2.at[1]], rows1, sem1)

                    @pl.loop(0, cnt_s // 2)
                    def _(kk):
                        k0 = kk * 2
                        pltpu.make_async_copy(h_hbm.at[si2.at[0]], rows0,
                                              sem0).wait()
                        pltpu.sync_copy(rows0, table.at[di2.at[0]], add=True)

                        @pl.when(k0 + 2 < cnt_s)
                        def _():
                            unpack(k0 + 2, 0)
                            pltpu.async_copy(h_hbm.at[si2.at[0]], rows0, sem0)

                        pltpu.make_async_copy(h_hbm.at[si2.at[1]], rows1,
                                              sem1).wait()
                        pltpu.sync_copy(rows1, table.at[di2.at[1]], add=True)

                        @pl.when(k0 + 3 < cnt_s)
                        def _():
                            unpack(k0 + 3, 1)
                            pltpu.async_copy(h_hbm.at[si2.at[1]], rows1, sem1)

                    @pl.when(cnt_s % 2 == 1)
                    def _():
                        pltpu.make_async_copy(h_hbm.at[si2.at[0]], rows0,
                                              sem0).wait()
                        pltpu.sync_copy(rows0, table.at[di2.at[0]], add=True)

        plsc.subcore_barrier()
        pltpu.sync_copy(table.at[pl.ds(sid * zr, zr)],
                        out_hbm.at[pl.ds(cid * TH + sid * zr, zr)])

    return _sc_agg


_SC_AGG = {j: _make_sc_agg(j) for j in range(NUM_CLUSTERS)}


def _update_body(j, agg_ref, h_ref, lab_ref, w1_ref, b1_ref, w2_ref, b2_ref, out_ref):
    agg = jnp.concatenate(
        [agg_ref[0:H0, :], agg_ref[TH:TH + (N - H0), :]], axis=0)
    h = h_ref[...]
    z = agg + h
    hid = jnp.maximum(
        jnp.dot(z, w1_ref[...], preferred_element_type=jnp.float32) + b1_ref[...], 0.0)
    new = jnp.dot(hid, w2_ref[...], preferred_element_type=jnp.float32) + b2_ref[...]
    mask = lab_ref[...] == j
    out_ref[...] = jnp.where(mask, new, h)


def _tc_update(j, agg2, h, labels, W1, b1, W2, b2):
    return pl.pallas_call(
        functools.partial(_update_body, j),
        out_shape=jax.ShapeDtypeStruct((N, D), jnp.float32),
    )(agg2, h, labels, W1, b1, W2, b2)


def _pool_body(h_ref, batch_ref, w1_ref, b1_ref, w2_ref, b2_ref, out_ref):
    rows = lax.broadcasted_iota(jnp.int32, (NUM_GRAPHS, N), 0)
    onehot = (rows == batch_ref[...]).astype(jnp.float32)
    pooled = jnp.dot(onehot, h_ref[...], preferred_element_type=jnp.float32)
    hid = jnp.maximum(
        jnp.dot(pooled, w1_ref[...], preferred_element_type=jnp.float32) + b1_ref[...], 0.0)
    out_ref[...] = jnp.dot(hid, w2_ref[...], preferred_element_type=jnp.float32) + b2_ref[...]


def _pool(h, batch_row, W1, b1, W2, b2):
    return pl.pallas_call(
        _pool_body,
        out_shape=jax.ShapeDtypeStruct((NUM_GRAPHS, D), jnp.float32),
    )(h, batch_row, W1, b1, W2, b2)


def kernel(x, conv_W1, conv_b1, conv_W2, conv_b2,
           pool_W1, pool_b1, pool_W2, pool_b2,
           cluster_labels, edge_index, batch):
    src = edge_index[0].astype(jnp.int32)
    dst = edge_index[1].astype(jnp.int32)
    lab1d = cluster_labels.astype(jnp.int32)
    labels = lab1d.reshape(N, 1)
    batch_row = batch.astype(jnp.int32).reshape(1, N)

    bpacked, cnts = _sc_part(src, dst, lab1d)

    h = x
    for i in range(NUM_LAYERS):
        for j in range(NUM_CLUSTERS):
            idx = i * NUM_CLUSTERS + j
            agg2 = _SC_AGG[j](h, bpacked, cnts)
            h = _tc_update(j, agg2, h, labels,
                           conv_W1[idx], conv_b1[idx].reshape(1, D),
                           conv_W2[idx], conv_b2[idx].reshape(1, D))
    return _pool(h, batch_row, pool_W1, pool_b1.reshape(1, D),
                 pool_W2, pool_b2.reshape(1, D))


# speedup vs baseline: 26.2567x; 1.0267x over previous
import dataclasses
import functools

import jax
import jax.numpy as jnp
from jax import lax
from jax.experimental import pallas as pl
from jax.experimental.pallas import tpu as pltpu
from jax.experimental.pallas import tpu_sc as plsc

N = 10000
E = 320000
D = 128
NUM_LAYERS = 2
NUM_CLUSTERS = 4
NUM_GRAPHS = 16

H0 = 5120
TH = 5248
CHUNK = 128
NWORK = 32
EW = E // NWORK
BLK = 2000
CAPC = 80
CAP = CAPC * CHUNK
NBKT = 2 * NUM_CLUSTERS
NREG = NBKT * NWORK
PSH = 8192

_mesh = plsc.VectorSubcoreMesh(core_axis_name="c", subcore_axis_name="s")

_cp = pltpu.CompilerParams()
if "needs_layout_passes" in pltpu.CompilerParams.__dataclass_fields__:
    _cp = dataclasses.replace(_cp, needs_layout_passes=False)


@functools.partial(
    pl.kernel,
    out_type=(
        jax.ShapeDtypeStruct((NREG * CAP,), jnp.int32),
        jax.ShapeDtypeStruct((NWORK, 16), jnp.int32),
    ),
    mesh=_mesh,
    scratch_types=[
        pltpu.VMEM((BLK,), jnp.int32),
        pltpu.VMEM((BLK,), jnp.int32),
        pltpu.VMEM((N,), jnp.int32),
        pltpu.VMEM((CAP,), jnp.int32),
        pltpu.VMEM((CAP,), jnp.int32),
        pltpu.VMEM((CAP,), jnp.int32),
        pltpu.VMEM((CAP,), jnp.int32),
        pltpu.VMEM((CAP,), jnp.int32),
        pltpu.VMEM((CAP,), jnp.int32),
        pltpu.VMEM((CAP,), jnp.int32),
        pltpu.VMEM((CAP,), jnp.int32),
        pltpu.VMEM((16,), jnp.int32),
    ],
    compiler_params=_cp,
)
def _sc_part(src_hbm, dst_hbm, lab_hbm, bp_hbm, cnt_hbm,
             sv, dv, lab, b0, b1, b2, b3, b4, b5, b6, b7, cv):
    cid = lax.axis_index("c")
    sid = lax.axis_index("s")
    w = sid * 2 + cid
    bb = (b0, b1, b2, b3, b4, b5, b6, b7)
    pltpu.sync_copy(lab_hbm, lab)
    lane = lax.iota(jnp.int32, 16)

    cnt = tuple(jnp.int32(0) for _ in range(NBKT))
    for blk in range(EW // BLK):
        pltpu.sync_copy(src_hbm.at[pl.ds(w * EW + blk * BLK, BLK)], sv)
        pltpu.sync_copy(dst_hbm.at[pl.ds(w * EW + blk * BLK, BLK)], dv)

        def body(v, c8):
            s16 = sv[pl.ds(v * 16, 16)]
            d16 = dv[pl.ds(v * 16, 16)]
            k16 = plsc.load_gather(lab, [d16])
            h16 = (d16 >= H0).astype(jnp.int32)
            b16 = k16 * 2 + h16
            p16 = s16 * PSH + (d16 - h16 * H0)
            out = []
            for b in range(NBKT):
                m = b16 == b
                plsc.store_compressed(bb[b].at[pl.ds(c8[b], 16)], p16, mask=m)
                nb_ = jnp.max(plsc.all_reduce_population_count(m))
                out.append(c8[b] + nb_)
            return tuple(out)

        cnt = lax.fori_loop(0, BLK // 16, body, cnt)

    cvec = jnp.zeros((16,), jnp.int32)
    for b in range(NBKT):
        cb = cnt[b]
        for t in range(8):
            off = cb + t * 16
            pad = lane + (t * 16 + w * 128)
            bb[b][pl.ds(off, 16)] = (pad % N) * PSH + H0 + pad % (TH - H0)
        nch = (cb + CHUNK) // CHUNK
        cvec = jnp.where(lane == b, jnp.broadcast_to(nch, (16,)), cvec)
        base = (b * NWORK + w) * CAP
        nblk = (cb + CHUNK + 1023) // 1024

        @pl.loop(0, nblk)
        def _(q):
            pltpu.sync_copy(bb[b].at[pl.ds(q * 1024, 1024)],
                            bp_hbm.at[pl.ds(base + q * 1024, 1024)])

    cv[...] = cvec
    pltpu.sync_copy(cv, cnt_hbm.at[w])


def _make_sc_agg(j):
    @functools.partial(
        pl.kernel,
        out_type=jax.ShapeDtypeStruct((2 * TH, D), jnp.float32),
        mesh=_mesh,
        scratch_types=[
            pltpu.VMEM((CAP // 2,), jnp.int32),
            pltpu.VMEM((2, CHUNK), jnp.int32),
            pltpu.VMEM((2, CHUNK), jnp.int32),
            pltpu.VMEM((CHUNK, D), jnp.float32),
            pltpu.VMEM((CHUNK, D), jnp.float32),
            pltpu.VMEM((CHUNK, D), jnp.float32),
            pltpu.VMEM((16,), jnp.int32),
            pltpu.VMEM((16,), jnp.int32),
            pltpu.VMEM_SHARED((TH, D), jnp.float32),
            pltpu.SemaphoreType.DMA,
            pltpu.SemaphoreType.DMA,
        ],
        compiler_params=_cp,
    )
    def _sc_agg(h_hbm, bp_hbm, cnt_hbm, out_hbm,
                pb, si2, di2, rows0, rows1, zbuf, cv0, cv1, table, sem0, sem1):
        cid = lax.axis_index("c")
        sid = lax.axis_index("s")
        HALF = CAPC // 2
        b = j * 2 + cid
        lane = lax.iota(jnp.int32, 16)
        zr = TH // 16

        pltpu.sync_copy(cnt_hbm.at[sid * 2], cv0)
        nch0 = jnp.max(jnp.where(lane == b, cv0[...], 0))
        cnt00 = jnp.clip(nch0, 0, HALF)
        base0 = (b * NWORK + sid * 2) * CAP

        def unpack(k, slot):
            for t in range(CHUNK // 16):
                pv = pb[pl.ds(k * CHUNK + t * 16, 16)]
                si2.at[slot][pl.ds(t * 16, 16)] = lax.shift_right_logical(
                    pv, jnp.int32(13))
                di2.at[slot][pl.ds(t * 16, 16)] = lax.bitwise_and(
                    pv, jnp.int32(PSH - 1))

        @pl.when(cnt00 > 0)
        def _():
            pltpu.sync_copy(bp_hbm.at[pl.ds(base0, CAP // 2)], pb)
            unpack(0, 0)
            pltpu.async_copy(h_hbm.at[si2.at[0]], rows0, sem0)

            @pl.when(cnt00 > 1)
            def _():
                unpack(1, 1)
                pltpu.async_copy(h_hbm.at[si2.at[1]], rows1, sem1)

        z16 = jnp.zeros((16,), jnp.float32)

        @pl.loop(0, CHUNK)
        def _(r):
            for t in range(D // 16):
                zbuf[r, pl.ds(t * 16, 16)] = z16

        pltpu.sync_copy(zbuf, table.at[pl.ds(sid * zr, CHUNK)])
        pltpu.sync_copy(zbuf, table.at[pl.ds(sid * zr + CHUNK, CHUNK)])
        pltpu.sync_copy(zbuf.at[pl.ds(0, zr - 2 * CHUNK)],
                        table.at[pl.ds(sid * zr + 2 * CHUNK, zr - 2 * CHUNK)])
        plsc.subcore_barrier()

        for r2 in range(2):
            r = sid * 2 + r2
            if r2 == 0:
                nch = nch0
            else:
                pltpu.sync_copy(cnt_hbm.at[r], cv1)
                nch = jnp.max(jnp.where(lane == b, cv1[...], 0))
            base = (b * NWORK + r) * CAP

            for s in range(2):
                cnt_s = jnp.clip(nch - s * HALF, 0, HALF)

                @pl.when(cnt_s > 0)
                def _():
                    if not (r2 == 0 and s == 0):
                        pltpu.sync_copy(bp_hbm.at[pl.ds(base + s * (CAP // 2),
                                                        CAP // 2)], pb)
                        unpack(0, 0)
                        pltpu.async_copy(h_hbm.at[si2.at[0]], rows0, sem0)

                        @pl.when(cnt_s > 1)
                        def _():
                            unpack(1, 1)
                            pltpu.async_copy(h_hbm.at[si2.at[1]], rows1, sem1)

                    @pl.loop(0, cnt_s // 2)
                    def _(kk):
                        k0 = kk * 2
                        pltpu.make_async_copy(h_hbm.at[si2.at[0]], rows0,
                                              sem0).wait()
                        pltpu.sync_copy(rows0, table.at[di2.at[0]], add=True)

                        @pl.when(k0 + 2 < cnt_s)
                        def _():
                            unpack(k0 + 2, 0)
                            pltpu.async_copy(h_hbm.at[si2.at[0]], rows0, sem0)

                        pltpu.make_async_copy(h_hbm.at[si2.at[1]], rows1,
                                              sem1).wait()
                        pltpu.sync_copy(rows1, table.at[di2.at[1]], add=True)

                        @pl.when(k0 + 3 < cnt_s)
                        def _():
                            unpack(k0 + 3, 1)
                            pltpu.async_copy(h_hbm.at[si2.at[1]], rows1, sem1)

                    @pl.when(cnt_s % 2 == 1)
                    def _():
                        pltpu.make_async_copy(h_hbm.at[si2.at[0]], rows0,
                                              sem0).wait()
                        pltpu.sync_copy(rows0, table.at[di2.at[0]], add=True)

        plsc.subcore_barrier()
        pltpu.sync_copy(table.at[pl.ds(sid * zr, zr)],
                        out_hbm.at[pl.ds(cid * TH + sid * zr, zr)])

    return _sc_agg


_SC_AGG = {j: _make_sc_agg(j) for j in range(NUM_CLUSTERS)}


def _update_body(j, agg_ref, h_ref, lab_ref, w1_ref, b1_ref, w2_ref, b2_ref, out_ref):
    agg = jnp.concatenate(
        [agg_ref[0:H0, :], agg_ref[TH:TH + (N - H0), :]], axis=0)
    h = h_ref[...]
    z = agg + h
    hid = jnp.maximum(
        jnp.dot(z, w1_ref[...], preferred_element_type=jnp.float32) + b1_ref[...], 0.0)
    new = jnp.dot(hid, w2_ref[...], preferred_element_type=jnp.float32) + b2_ref[...]
    mask = lab_ref[...] == j
    out_ref[...] = jnp.where(mask, new, h)


def _tc_update(j, agg2, h, labels, W1, b1, W2, b2):
    return pl.pallas_call(
        functools.partial(_update_body, j),
        out_shape=jax.ShapeDtypeStruct((N, D), jnp.float32),
    )(agg2, h, labels, W1, b1, W2, b2)


def _pool_body(h_ref, batch_ref, w1_ref, b1_ref, w2_ref, b2_ref, out_ref):
    rows = lax.broadcasted_iota(jnp.int32, (NUM_GRAPHS, N), 0)
    onehot = (rows == batch_ref[...]).astype(jnp.float32)
    pooled = jnp.dot(onehot, h_ref[...], preferred_element_type=jnp.float32)
    hid = jnp.maximum(
        jnp.dot(pooled, w1_ref[...], preferred_element_type=jnp.float32) + b1_ref[...], 0.0)
    out_ref[...] = jnp.dot(hid, w2_ref[...], preferred_element_type=jnp.float32) + b2_ref[...]


def _pool(h, batch_row, W1, b1, W2, b2):
    return pl.pallas_call(
        _pool_body,
        out_shape=jax.ShapeDtypeStruct((NUM_GRAPHS, D), jnp.float32),
    )(h, batch_row, W1, b1, W2, b2)


def kernel(x, conv_W1, conv_b1, conv_W2, conv_b2,
           pool_W1, pool_b1, pool_W2, pool_b2,
           cluster_labels, edge_index, batch):
    src = edge_index[0].astype(jnp.int32)
    dst = edge_index[1].astype(jnp.int32)
    lab1d = cluster_labels.astype(jnp.int32)
    labels = lab1d.reshape(N, 1)
    batch_row = batch.astype(jnp.int32).reshape(1, N)

    bpacked, cnts = _sc_part(src, dst, lab1d)

    h = x
    for i in range(NUM_LAYERS):
        for j in range(NUM_CLUSTERS):
            idx = i * NUM_CLUSTERS + j
            agg2 = _SC_AGG[j](h, bpacked, cnts)
            h = _tc_update(j, agg2, h, labels,
                           conv_W1[idx], conv_b1[idx].reshape(1, D),
                           conv_W2[idx], conv_b2[idx].reshape(1, D))
    return _pool(h, batch_row, pool_W1, pool_b1.reshape(1, D),
                 pool_W2, pool_b2.reshape(1, D))


# speedup vs baseline: 27.2142x; 1.0365x over previous
import dataclasses
import functools

import jax
import jax.numpy as jnp
from jax import lax
from jax.experimental import pallas as pl
from jax.experimental.pallas import tpu as pltpu
from jax.experimental.pallas import tpu_sc as plsc

N = 10000
E = 320000
D = 128
NUM_LAYERS = 2
NUM_CLUSTERS = 4
NUM_GRAPHS = 16

H0 = 5120
TH = 5248
CHUNK = 128
NWORK = 32
EW = E // NWORK
BLK = 2000
CAPC = 80
CAP = CAPC * CHUNK
NBKT = 2 * NUM_CLUSTERS
NREG = NBKT * NWORK
PSH = 8192

_mesh = plsc.VectorSubcoreMesh(core_axis_name="c", subcore_axis_name="s")

_cp = pltpu.CompilerParams()
if "needs_layout_passes" in pltpu.CompilerParams.__dataclass_fields__:
    _cp = dataclasses.replace(_cp, needs_layout_passes=False)


@functools.partial(
    pl.kernel,
    out_type=(
        jax.ShapeDtypeStruct((NREG * CAP,), jnp.int32),
        jax.ShapeDtypeStruct((NWORK, 16), jnp.int32),
    ),
    mesh=_mesh,
    scratch_types=[
        pltpu.VMEM((BLK,), jnp.int32),
        pltpu.VMEM((BLK,), jnp.int32),
        pltpu.VMEM((N,), jnp.int32),
        pltpu.VMEM((CAP,), jnp.int32),
        pltpu.VMEM((CAP,), jnp.int32),
        pltpu.VMEM((CAP,), jnp.int32),
        pltpu.VMEM((CAP,), jnp.int32),
        pltpu.VMEM((CAP,), jnp.int32),
        pltpu.VMEM((CAP,), jnp.int32),
        pltpu.VMEM((CAP,), jnp.int32),
        pltpu.VMEM((CAP,), jnp.int32),
        pltpu.VMEM((16,), jnp.int32),
    ],
    compiler_params=_cp,
)
def _sc_part(src_hbm, dst_hbm, lab_hbm, bp_hbm, cnt_hbm,
             sv, dv, lab, b0, b1, b2, b3, b4, b5, b6, b7, cv):
    cid = lax.axis_index("c")
    sid = lax.axis_index("s")
    w = sid * 2 + cid
    bb = (b0, b1, b2, b3, b4, b5, b6, b7)
    pltpu.sync_copy(lab_hbm, lab)
    lane = lax.iota(jnp.int32, 16)

    cnt = tuple(jnp.int32(0) for _ in range(NBKT))
    for blk in range(EW // BLK):
        pltpu.sync_copy(src_hbm.at[pl.ds(w * EW + blk * BLK, BLK)], sv)
        pltpu.sync_copy(dst_hbm.at[pl.ds(w * EW + blk * BLK, BLK)], dv)

        def body(v, c8):
            s16 = sv[pl.ds(v * 16, 16)]
            d16 = dv[pl.ds(v * 16, 16)]
            k16 = plsc.load_gather(lab, [d16])
            h16 = (d16 >= H0).astype(jnp.int32)
            b16 = k16 * 2 + h16
            p16 = s16 * PSH + (d16 - h16 * H0)
            out = []
            for b in range(NBKT):
                m = b16 == b
                plsc.store_compressed(bb[b].at[pl.ds(c8[b], 16)], p16, mask=m)
                nb_ = jnp.max(plsc.all_reduce_population_count(m))
                out.append(c8[b] + nb_)
            return tuple(out)

        cnt = lax.fori_loop(0, BLK // 16, body, cnt)

    cvec = jnp.zeros((16,), jnp.int32)
    for b in range(NBKT):
        cb = cnt[b]
        for t in range(8):
            off = cb + t * 16
            pad = lane + (t * 16 + w * 128)
            bb[b][pl.ds(off, 16)] = (pad % N) * PSH + H0 + pad % (TH - H0)
        nch = (cb + CHUNK) // CHUNK
        cvec = jnp.where(lane == b, jnp.broadcast_to(nch, (16,)), cvec)
        base = (b * NWORK + w) * CAP
        nblk = (cb + CHUNK + 1023) // 1024

        @pl.loop(0, nblk)
        def _(q):
            pltpu.sync_copy(bb[b].at[pl.ds(q * 1024, 1024)],
                            bp_hbm.at[pl.ds(base + q * 1024, 1024)])

    cv[...] = cvec
    pltpu.sync_copy(cv, cnt_hbm.at[w])


def _make_sc_agg(j):
    @functools.partial(
        pl.kernel,
        out_type=jax.ShapeDtypeStruct((2 * TH, D), jnp.float32),
        mesh=_mesh,
        scratch_types=[
            pltpu.VMEM((CAP // 2,), jnp.int32),
            pltpu.VMEM((3, CHUNK), jnp.int32),
            pltpu.VMEM((3, CHUNK), jnp.int32),
            pltpu.VMEM((CHUNK, D), jnp.float32),
            pltpu.VMEM((CHUNK, D), jnp.float32),
            pltpu.VMEM((CHUNK, D), jnp.float32),
            pltpu.VMEM((CHUNK, D), jnp.float32),
            pltpu.VMEM((16,), jnp.int32),
            pltpu.VMEM((16,), jnp.int32),
            pltpu.VMEM_SHARED((TH, D), jnp.float32),
            pltpu.SemaphoreType.DMA,
            pltpu.SemaphoreType.DMA,
            pltpu.SemaphoreType.DMA,
        ],
        compiler_params=_cp,
    )
    def _sc_agg(h_hbm, bp_hbm, cnt_hbm, out_hbm,
                pb, si2, di2, rows0, rows1, rows2, zbuf, cv0, cv1, table,
                sem0, sem1, sem2):
        cid = lax.axis_index("c")
        sid = lax.axis_index("s")
        HALF = CAPC // 2
        b = j * 2 + cid
        lane = lax.iota(jnp.int32, 16)
        zr = TH // 16

        pltpu.sync_copy(cnt_hbm.at[sid * 2], cv0)
        nch0 = jnp.max(jnp.where(lane == b, cv0[...], 0))
        cnt00 = jnp.clip(nch0, 0, HALF)
        base0 = (b * NWORK + sid * 2) * CAP

        def unpack(k, slot):
            for t in range(CHUNK // 16):
                pv = pb[pl.ds(k * CHUNK + t * 16, 16)]
                si2.at[slot][pl.ds(t * 16, 16)] = lax.shift_right_logical(
                    pv, jnp.int32(13))
                di2.at[slot][pl.ds(t * 16, 16)] = lax.bitwise_and(
                    pv, jnp.int32(PSH - 1))

        ROWS = (rows0, rows1, rows2)
        SEMS = (sem0, sem1, sem2)

        def prime(cnt_s):
            for u in range(3):
                @pl.when(cnt_s > u)
                def _(u=u):
                    unpack(u, u)
                    pltpu.async_copy(h_hbm.at[si2.at[u]], ROWS[u], SEMS[u])

        def drain(cnt_s):
            @pl.loop(0, cnt_s // 3)
            def _(kk):
                k0 = kk * 3
                for u in range(3):
                    pltpu.make_async_copy(h_hbm.at[si2.at[u]], ROWS[u],
                                          SEMS[u]).wait()
                    pltpu.sync_copy(ROWS[u], table.at[di2.at[u]], add=True)

                    @pl.when(k0 + u + 3 < cnt_s)
                    def _(u=u):
                        unpack(k0 + u + 3, u)
                        pltpu.async_copy(h_hbm.at[si2.at[u]], ROWS[u], SEMS[u])

            rem = cnt_s % 3
            for u in range(2):
                @pl.when(rem > u)
                def _(u=u):
                    pltpu.make_async_copy(h_hbm.at[si2.at[u]], ROWS[u],
                                          SEMS[u]).wait()
                    pltpu.sync_copy(ROWS[u], table.at[di2.at[u]], add=True)

        @pl.when(cnt00 > 0)
        def _():
            pltpu.sync_copy(bp_hbm.at[pl.ds(base0, CAP // 2)], pb)
            prime(cnt00)

        z16 = jnp.zeros((16,), jnp.float32)

        @pl.loop(0, CHUNK)
        def _(r):
            for t in range(D // 16):
                zbuf[r, pl.ds(t * 16, 16)] = z16

        pltpu.sync_copy(zbuf, table.at[pl.ds(sid * zr, CHUNK)])
        pltpu.sync_copy(zbuf, table.at[pl.ds(sid * zr + CHUNK, CHUNK)])
        pltpu.sync_copy(zbuf.at[pl.ds(0, zr - 2 * CHUNK)],
                        table.at[pl.ds(sid * zr + 2 * CHUNK, zr - 2 * CHUNK)])
        plsc.subcore_barrier()

        for r2 in range(2):
            r = sid * 2 + r2
            if r2 == 0:
                nch = nch0
            else:
                pltpu.sync_copy(cnt_hbm.at[r], cv1)
                nch = jnp.max(jnp.where(lane == b, cv1[...], 0))
            base = (b * NWORK + r) * CAP

            for s in range(2):
                cnt_s = jnp.clip(nch - s * HALF, 0, HALF)

                @pl.when(cnt_s > 0)
                def _():
                    if not (r2 == 0 and s == 0):
                        pltpu.sync_copy(bp_hbm.at[pl.ds(base + s * (CAP // 2),
                                                        CAP // 2)], pb)
                        prime(cnt_s)
                    drain(cnt_s)

        plsc.subcore_barrier()
        pltpu.sync_copy(table.at[pl.ds(sid * zr, zr)],
                        out_hbm.at[pl.ds(cid * TH + sid * zr, zr)])

    return _sc_agg


_SC_AGG = {j: _make_sc_agg(j) for j in range(NUM_CLUSTERS)}


def _update_body(j, agg_ref, h_ref, lab_ref, w1_ref, b1_ref, w2_ref, b2_ref, out_ref):
    agg = jnp.concatenate(
        [agg_ref[0:H0, :], agg_ref[TH:TH + (N - H0), :]], axis=0)
    h = h_ref[...]
    z = agg + h
    hid = jnp.maximum(
        jnp.dot(z, w1_ref[...], preferred_element_type=jnp.float32) + b1_ref[...], 0.0)
    new = jnp.dot(hid, w2_ref[...], preferred_element_type=jnp.float32) + b2_ref[...]
    mask = lab_ref[...] == j
    out_ref[...] = jnp.where(mask, new, h)


def _tc_update(j, agg2, h, labels, W1, b1, W2, b2):
    return pl.pallas_call(
        functools.partial(_update_body, j),
        out_shape=jax.ShapeDtypeStruct((N, D), jnp.float32),
    )(agg2, h, labels, W1, b1, W2, b2)


def _pool_body(h_ref, batch_ref, w1_ref, b1_ref, w2_ref, b2_ref, out_ref):
    rows = lax.broadcasted_iota(jnp.int32, (NUM_GRAPHS, N), 0)
    onehot = (rows == batch_ref[...]).astype(jnp.float32)
    pooled = jnp.dot(onehot, h_ref[...], preferred_element_type=jnp.float32)
    hid = jnp.maximum(
        jnp.dot(pooled, w1_ref[...], preferred_element_type=jnp.float32) + b1_ref[...], 0.0)
    out_ref[...] = jnp.dot(hid, w2_ref[...], preferred_element_type=jnp.float32) + b2_ref[...]


def _pool(h, batch_row, W1, b1, W2, b2):
    return pl.pallas_call(
        _pool_body,
        out_shape=jax.ShapeDtypeStruct((NUM_GRAPHS, D), jnp.float32),
    )(h, batch_row, W1, b1, W2, b2)


def kernel(x, conv_W1, conv_b1, conv_W2, conv_b2,
           pool_W1, pool_b1, pool_W2, pool_b2,
           cluster_labels, edge_index, batch):
    src = edge_index[0].astype(jnp.int32)
    dst = edge_index[1].astype(jnp.int32)
    lab1d = cluster_labels.astype(jnp.int32)
    labels = lab1d.reshape(N, 1)
    batch_row = batch.astype(jnp.int32).reshape(1, N)

    bpacked, cnts = _sc_part(src, dst, lab1d)

    h = x
    for i in range(NUM_LAYERS):
        for j in range(NUM_CLUSTERS):
            idx = i * NUM_CLUSTERS + j
            agg2 = _SC_AGG[j](h, bpacked, cnts)
            h = _tc_update(j, agg2, h, labels,
                           conv_W1[idx], conv_b1[idx].reshape(1, D),
                           conv_W2[idx], conv_b2[idx].reshape(1, D))
    return _pool(h, batch_row, pool_W1, pool_b1.reshape(1, D),
                 pool_W2, pool_b2.reshape(1, D))
